# baseline (speedup 1.0000x reference)
import functools
import math

import jax
import jax.numpy as jnp
import numpy as np
from jax import lax
from jax.experimental import pallas as pl
from jax.experimental.pallas import tpu as pltpu

HID = 2048; NH = 16; DK = 128; DR = 64; DV = 128; DC = 512; DCP = 1536
INH = 8; IHD = 128; ITOPK = 8; MAXS = 2048
SEQ = 2048

_NEG = -1e30


def _mm_kernel(a_ref, b_ref, o_ref):
    o_ref[...] = jnp.dot(a_ref[...], b_ref[...],
                         preferred_element_type=jnp.float32)


def _matmul(a, b, bm, bn):
    M, K = a.shape
    N = b.shape[1]
    return pl.pallas_call(
        _mm_kernel,
        grid=(M // bm, N // bn),
        in_specs=[
            pl.BlockSpec((bm, K), lambda i, j: (i, 0)),
            pl.BlockSpec((K, bn), lambda i, j: (0, j)),
        ],
        out_specs=pl.BlockSpec((bm, bn), lambda i, j: (i, j)),
        out_shape=jax.ShapeDtypeStruct((M, N), jnp.float32),
        compiler_params=pltpu.CompilerParams(
            dimension_semantics=("parallel", "parallel")),
    )(a.astype(jnp.bfloat16), b.astype(jnp.bfloat16))


def _mm_bias_kernel(a_ref, bias_ref, b_ref, o_ref):
    s = a_ref[...] + bias_ref[...]
    o_ref[...] = jnp.dot(s.astype(jnp.bfloat16), b_ref[...],
                         preferred_element_type=jnp.float32)


def _matmul_bias(a, bias, b, bm, bn):
    M, K = a.shape
    N = b.shape[1]
    return pl.pallas_call(
        _mm_bias_kernel,
        grid=(M // bm, N // bn),
        in_specs=[
            pl.BlockSpec((bm, K), lambda i, j: (i, 0)),
            pl.BlockSpec((1, K), lambda i, j: (0, 0)),
            pl.BlockSpec((K, bn), lambda i, j: (0, j)),
        ],
        out_specs=pl.BlockSpec((bm, bn), lambda i, j: (i, j)),
        out_shape=jax.ShapeDtypeStruct((M, N), jnp.float32),
        compiler_params=pltpu.CompilerParams(
            dimension_semantics=("parallel", "parallel")),
    )(a, bias, b.astype(jnp.bfloat16))


def _flash_kernel(qc_ref, qr_ref, kc_ref, kr_ref, v_ref, o_ref, *, bq, bk):
    iq = pl.program_id(1)
    scale = 1.0 / math.sqrt(DK + DR)
    qc = qc_ref[...]
    qr = qr_ref[0]

    def body(j, carry):
        m, l, acc = carry
        kc = kc_ref[pl.ds(j * bk, bk), :]
        kr = kr_ref[pl.ds(j * bk, bk), :]
        vb = v_ref[pl.ds(j * bk, bk), :]
        s = lax.dot_general(qc, kc, (((1,), (1,)), ((), ())),
                            preferred_element_type=jnp.float32)
        s += lax.dot_general(qr, kr, (((1,), (1,)), ((), ())),
                             preferred_element_type=jnp.float32)
        s *= scale
        row = iq * bq + lax.broadcasted_iota(jnp.int32, (bq, bk), 0)
        col = j * bk + lax.broadcasted_iota(jnp.int32, (bq, bk), 1)
        s = jnp.where(col <= row, s, _NEG)
        m_new = jnp.maximum(m, jnp.max(s, axis=1, keepdims=True))
        p = jnp.exp(s - m_new)
        alpha = jnp.exp(m - m_new)
        l_new = l * alpha + jnp.sum(p, axis=1, keepdims=True)
        acc_new = acc * alpha + jnp.dot(p.astype(jnp.bfloat16), vb,
                                        preferred_element_type=jnp.float32)
        return m_new, l_new, acc_new

    m0 = jnp.full((bq, 1), _NEG, jnp.float32)
    l0 = jnp.zeros((bq, 1), jnp.float32)
    acc0 = jnp.zeros((bq, DV), jnp.float32)
    nj = (iq + 1) * bq // bk
    m, l, acc = lax.fori_loop(0, nj, body, (m0, l0, acc0))
    o_ref[...] = acc / l


def _flash(qc, qr, kc, kr, v, bq, bk):
    kern = functools.partial(_flash_kernel, bq=bq, bk=bk)
    return pl.pallas_call(
        kern,
        grid=(NH, SEQ // bq),
        in_specs=[
            pl.BlockSpec((bq, DK), lambda h, i: (i, h)),
            pl.BlockSpec((1, bq, DR), lambda h, i: (h, i, 0)),
            pl.BlockSpec((SEQ, DK), lambda h, i: (0, h)),
            pl.BlockSpec((SEQ, DR), lambda h, i: (0, 0)),
            pl.BlockSpec((SEQ, DV), lambda h, i: (0, h)),
        ],
        out_specs=pl.BlockSpec((bq, DV), lambda h, i: (i, h)),
        out_shape=jax.ShapeDtypeStruct((SEQ, NH * DV), jnp.float32),
        compiler_params=pltpu.CompilerParams(
            dimension_semantics=("parallel", "parallel")),
    )(qc.astype(jnp.bfloat16), qr.astype(jnp.bfloat16),
      kc.astype(jnp.bfloat16), kr.astype(jnp.bfloat16),
      v.astype(jnp.bfloat16))


def _indexer_kernel(g_ref, ip_ref, wq_ref, wk_ref, wv_ref, wo_ref, o_ref):
    g = g_ref[...]
    iota = lax.broadcasted_iota(jnp.int32, (INH, SEQ), 1)
    ip = ip_ref[...]
    dm = (lax.broadcasted_iota(jnp.int32, (INH, INH, 1), 0)
          == lax.broadcasted_iota(jnp.int32, (INH, INH, 1), 1))
    sel_rows = []
    for _ in range(ITOPK):
        m = jnp.max(g, axis=1, keepdims=True)
        cand = jnp.where(g >= m, iota, SEQ)
        idx = jnp.min(cand, axis=1, keepdims=True)
        hot = iota == idx
        onehot = hot.astype(jnp.float32)
        g = jnp.where(hot, _NEG, g)
        full = jnp.dot(onehot, ip, preferred_element_type=jnp.float32)
        f3 = full.reshape(INH, INH, IHD)
        sel_rows.append(jnp.sum(jnp.where(dm, f3, 0.0), axis=1))
    st = jnp.concatenate(sel_rows, axis=0)
    sq = jnp.dot(st, wq_ref[...], preferred_element_type=jnp.float32)
    sk = jnp.dot(st, wk_ref[...], preferred_element_type=jnp.float32)
    sv = jnp.dot(st, wv_ref[...], preferred_element_type=jnp.float32)
    sc = lax.dot_general(sq, sk, (((1,), (1,)), ((), ())),
                         preferred_element_type=jnp.float32)
    sc *= 1.0 / math.sqrt(IHD)
    mx = jnp.max(sc, axis=1, keepdims=True)
    e = jnp.exp(sc - mx)
    p = e / jnp.sum(e, axis=1, keepdims=True)
    so = jnp.dot(p, sv, preferred_element_type=jnp.float32)
    spo = jnp.dot(so, wo_ref[...], preferred_element_type=jnp.float32)
    o_ref[...] = jnp.mean(spo.reshape(ITOPK, INH, DV), axis=0)


def _indexer(gate_t, iproj, w_sq, w_sk, w_sv, w_iout_dv):
    return pl.pallas_call(
        _indexer_kernel,
        out_shape=jax.ShapeDtypeStruct((INH, DV), jnp.float32),
    )(gate_t, iproj, w_sq, w_sk, w_sv, w_iout_dv)


def _rope_tables():
    inv = 1.0 / (10000.0 ** (np.arange(0, DR, 2)[: DR // 2]
                             .astype(np.float32) / DR))
    t = np.arange(SEQ, dtype=np.float32)
    f = np.outer(t, inv)
    return jnp.asarray(np.cos(f)), jnp.asarray(np.sin(f))


def _rope(x, cos, sin):
    x1 = x[..., 0::2]
    x2 = x[..., 1::2]
    c = cos[:, None, :]
    s = sin[:, None, :]
    o1 = x1 * c - x2 * s
    o2 = x1 * s + x2 * c
    return jnp.stack([o1, o2], axis=-1).reshape(x.shape)


def kernel(x, W_c, W_cp, W_qc, W_qr, W_kc, W_kr, W_v, W_o,
           W_iproj, W_igate, W_sq, W_sk, W_sv, W_iout):
    B = x.shape[0]
    x2 = x.reshape(B * SEQ, HID)

    w1 = jnp.concatenate([W_c, W_cp, W_kr, W_iproj, W_igate], axis=1)
    n1 = w1.shape[1]
    pad1 = (-n1) % 128
    w1 = jnp.pad(w1, ((0, 0), (0, pad1)))
    p1 = _matmul(x2, w1, bm=512, bn=640)
    c = p1[:, :DC]
    cp = p1[:, DC:DC + DCP]
    krp = p1[:, DC + DCP:DC + DCP + DR]
    iproj = p1[:, DC + DCP + DR:DC + DCP + DR + INH * IHD]
    gate = p1[:, DC + DCP + DR + INH * IHD:DC + DCP + DR + INH * IHD + INH]

    p2 = _matmul(cp, jnp.concatenate([W_qc, W_qr], axis=1), bm=512, bn=768)
    qc = p2[:, :NH * DK]
    qr = p2[:, NH * DK:]
    p3 = _matmul(c, jnp.concatenate([W_kc, W_v], axis=1), bm=512, bn=1024)
    kc = p3[:, :NH * DK]
    v = p3[:, NH * DK:]

    cos, sin = _rope_tables()
    qr = _rope(qr.reshape(SEQ, NH, DR), cos, sin).transpose(1, 0, 2)
    kr = _rope(krp.reshape(SEQ, 1, DR), cos, sin).reshape(SEQ, DR)

    attn = _flash(qc, qr, kc, kr, v, bq=512, bk=512)

    meanh = _indexer(gate.T, iproj, W_sq, W_sk, W_sv, W_iout[:, :DV])
    bias = jnp.repeat(meanh, NH // INH, axis=0).reshape(1, NH * DV)

    out = _matmul_bias(attn, bias, W_o, bm=512, bn=1024)
    return out.reshape(B, SEQ, HID)


# speedup vs baseline: 1.2767x; 1.2767x over previous
import functools
import math

import jax
import jax.numpy as jnp
import numpy as np
from jax import lax
from jax.experimental import pallas as pl
from jax.experimental.pallas import tpu as pltpu

HID = 2048; NH = 16; DK = 128; DR = 64; DV = 128; DC = 512; DCP = 1536
INH = 8; IHD = 128; ITOPK = 8; MAXS = 2048
SEQ = 2048

_NEG = -1e30

_P1N = 3200


def _proj1_kernel(x_ref, w_ref, wg_ref, p_ref, g_ref):
    xb = x_ref[...].astype(jnp.bfloat16)
    p_ref[...] = jnp.dot(xb, w_ref[...],
                         preferred_element_type=jnp.float32
                         ).astype(jnp.bfloat16)

    @pl.when(pl.program_id(0) == 0)
    def _():
        g_ref[...] = jnp.dot(x_ref[...], wg_ref[...],
                             preferred_element_type=jnp.float32)


def _proj1(x2, w1, wg, bn=640):
    return pl.pallas_call(
        _proj1_kernel,
        grid=(_P1N // bn,),
        in_specs=[
            pl.BlockSpec((SEQ, HID), lambda j: (0, 0)),
            pl.BlockSpec((HID, bn), lambda j: (0, j)),
            pl.BlockSpec((HID, INH), lambda j: (0, 0)),
        ],
        out_specs=[
            pl.BlockSpec((SEQ, bn), lambda j: (0, j)),
            pl.BlockSpec((SEQ, INH), lambda j: (0, 0)),
        ],
        out_shape=[
            jax.ShapeDtypeStruct((SEQ, _P1N), jnp.bfloat16),
            jax.ShapeDtypeStruct((SEQ, INH), jnp.float32),
        ],
    )(x2, w1, wg)


def _mmb_kernel(a_ref, b_ref, o_ref):
    o_ref[...] = jnp.dot(a_ref[...], b_ref[...],
                         preferred_element_type=jnp.float32
                         ).astype(jnp.bfloat16)


def _mmb(a_arr, a_block, a_idx, b, bn):
    K = a_block
    N = b.shape[1]
    return pl.pallas_call(
        _mmb_kernel,
        grid=(N // bn,),
        in_specs=[
            pl.BlockSpec((SEQ, K), lambda j: (0, a_idx)),
            pl.BlockSpec((K, bn), lambda j: (0, j)),
        ],
        out_specs=pl.BlockSpec((SEQ, bn), lambda j: (0, j)),
        out_shape=jax.ShapeDtypeStruct((SEQ, N), jnp.bfloat16),
    )(a_arr, b)


def _flash_kernel(qc_ref, qr_ref, kc_ref, kr_ref, v_ref, o_ref, *, bq, bk):
    iq = pl.program_id(1)
    qc = qc_ref[...]
    qr = qr_ref[0]

    def blk(j, masked):
        kc = kc_ref[pl.ds(j * bk, bk), :]
        kr = kr_ref[pl.ds(j * bk, bk), :]
        s = lax.dot_general(qc, kc, (((1,), (1,)), ((), ())),
                            preferred_element_type=jnp.float32)
        s += lax.dot_general(qr, kr, (((1,), (1,)), ((), ())),
                             preferred_element_type=jnp.float32)
        if masked:
            row = iq * bq + lax.broadcasted_iota(jnp.int32, (bq, bk), 0)
            col = j * bk + lax.broadcasted_iota(jnp.int32, (bq, bk), 1)
            s = jnp.where(col <= row, s, _NEG)
        return s, v_ref[pl.ds(j * bk, bk), :]

    def update(s, vb, carry):
        m, l, acc = carry
        m_new = jnp.maximum(m, jnp.max(s, axis=1, keepdims=True))
        p = jnp.exp(s - m_new)
        alpha = jnp.exp(m - m_new)
        l_new = l * alpha + jnp.sum(p, axis=1, keepdims=True)
        acc_new = acc * alpha + jnp.dot(p.astype(jnp.bfloat16), vb,
                                        preferred_element_type=jnp.float32)
        return m_new, l_new, acc_new

    def body(j, carry):
        s, vb = blk(j, masked=False)
        return update(s, vb, carry)

    m0 = jnp.full((bq, 1), _NEG, jnp.float32)
    l0 = jnp.zeros((bq, 1), jnp.float32)
    acc0 = jnp.zeros((bq, DV), jnp.float32)
    carry = lax.fori_loop(0, iq, body, (m0, l0, acc0))
    s, vb = blk(iq, masked=True)
    m, l, acc = update(s, vb, carry)
    o_ref[...] = (acc / l).astype(jnp.bfloat16)


def _flash(q_arr, qr_rope, kv_arr, kr_rope, bq, bk):
    kern = functools.partial(_flash_kernel, bq=bq, bk=bk)
    return pl.pallas_call(
        kern,
        grid=(NH, SEQ // bq),
        in_specs=[
            pl.BlockSpec((bq, DK), lambda h, i: (i, h)),
            pl.BlockSpec((1, bq, DR), lambda h, i: (h, i, 0)),
            pl.BlockSpec((SEQ, DK), lambda h, i: (0, h)),
            pl.BlockSpec((SEQ, DR), lambda h, i: (0, 0)),
            pl.BlockSpec((SEQ, DV), lambda h, i: (0, NH + h)),
        ],
        out_specs=pl.BlockSpec((bq, DV), lambda h, i: (i, h)),
        out_shape=jax.ShapeDtypeStruct((SEQ, NH * DV), jnp.bfloat16),
    )(q_arr, qr_rope, kv_arr, kr_rope, kv_arr)


def _indexer_kernel(g_ref, ip_ref, wq_ref, wk_ref, wv_ref, wo_ref, o_ref):
    g = g_ref[...]
    iota = lax.broadcasted_iota(jnp.int32, (INH, SEQ), 1)
    ip = ip_ref[...]
    dm = (lax.broadcasted_iota(jnp.int32, (INH, INH, 1), 0)
          == lax.broadcasted_iota(jnp.int32, (INH, INH, 1), 1))
    sel_rows = []
    for _ in range(ITOPK):
        m = jnp.max(g, axis=1, keepdims=True)
        cand = jnp.where(g >= m, iota, SEQ)
        idx = jnp.min(cand, axis=1, keepdims=True)
        hot = iota == idx
        onehot = hot.astype(jnp.bfloat16)
        g = jnp.where(hot, _NEG, g)
        full = jnp.dot(onehot, ip, preferred_element_type=jnp.float32)
        f3 = full.reshape(INH, INH, IHD)
        sel_rows.append(jnp.sum(jnp.where(dm, f3, 0.0), axis=1))
    st = jnp.concatenate(sel_rows, axis=0)
    sq = jnp.dot(st, wq_ref[...], preferred_element_type=jnp.float32)
    sk = jnp.dot(st, wk_ref[...], preferred_element_type=jnp.float32)
    sv = jnp.dot(st, wv_ref[...], preferred_element_type=jnp.float32)
    sc = lax.dot_general(sq, sk, (((1,), (1,)), ((), ())),
                         preferred_element_type=jnp.float32)
    sc *= 1.0 / math.sqrt(IHD)
    mx = jnp.max(sc, axis=1, keepdims=True)
    e = jnp.exp(sc - mx)
    p = e / jnp.sum(e, axis=1, keepdims=True)
    so = jnp.dot(p, sv, preferred_element_type=jnp.float32)
    spo = jnp.dot(so, wo_ref[...], preferred_element_type=jnp.float32)
    o_ref[...] = jnp.mean(spo.reshape(ITOPK, INH, DV), axis=0)


def _indexer(gate_t, p1, w_sq, w_sk, w_sv, w_iout_dv):
    return pl.pallas_call(
        _indexer_kernel,
        grid=(1,),
        in_specs=[
            pl.BlockSpec((INH, SEQ), lambda j: (0, 0)),
            pl.BlockSpec((SEQ, INH * IHD), lambda j: (0, 2)),
            pl.BlockSpec((IHD, IHD), lambda j: (0, 0)),
            pl.BlockSpec((IHD, IHD), lambda j: (0, 0)),
            pl.BlockSpec((IHD, IHD), lambda j: (0, 0)),
            pl.BlockSpec((IHD, DV), lambda j: (0, 0)),
        ],
        out_specs=pl.BlockSpec((INH, DV), lambda j: (0, 0)),
        out_shape=jax.ShapeDtypeStruct((INH, DV), jnp.float32),
    )(gate_t, p1, w_sq, w_sk, w_sv, w_iout_dv)


def _out_kernel(a_ref, bias_ref, b_ref, o_ref):
    s = (a_ref[...].astype(jnp.float32) + bias_ref[...]).astype(jnp.bfloat16)
    o_ref[...] = jnp.dot(s, b_ref[...].astype(jnp.bfloat16),
                         preferred_element_type=jnp.float32)


def _outproj(attn, bias, W_o, bn=1024):
    return pl.pallas_call(
        _out_kernel,
        grid=(HID // bn,),
        in_specs=[
            pl.BlockSpec((SEQ, NH * DV), lambda j: (0, 0)),
            pl.BlockSpec((1, NH * DV), lambda j: (0, 0)),
            pl.BlockSpec((NH * DV, bn), lambda j: (0, j)),
        ],
        out_specs=pl.BlockSpec((SEQ, bn), lambda j: (0, j)),
        out_shape=jax.ShapeDtypeStruct((SEQ, HID), jnp.float32),
    )(attn, bias, W_o)


def _rope_tables():
    inv = 1.0 / (10000.0 ** (np.arange(0, DR, 2)[: DR // 2]
                             .astype(np.float32) / DR))
    t = np.arange(SEQ, dtype=np.float32)
    f = np.outer(t, inv)
    return jnp.asarray(np.cos(f)), jnp.asarray(np.sin(f))


def _rope(x, cos, sin):
    x = x.astype(jnp.float32)
    x1 = x[..., 0::2]
    x2 = x[..., 1::2]
    c = cos[:, None, :]
    s = sin[:, None, :]
    o1 = x1 * c - x2 * s
    o2 = x1 * s + x2 * c
    return jnp.stack([o1, o2], axis=-1).reshape(x.shape)


def kernel(x, W_c, W_cp, W_qc, W_qr, W_kc, W_kr, W_v, W_o,
           W_iproj, W_igate, W_sq, W_sk, W_sv, W_iout):
    B = x.shape[0]
    x2 = x.reshape(B * SEQ, HID)
    bf = jnp.bfloat16

    w1 = jnp.concatenate([
        W_cp.astype(bf), W_c.astype(bf), W_iproj.astype(bf),
        W_kr.astype(bf), jnp.zeros((HID, _P1N - DCP - DC - INH * IHD - DR),
                                   dtype=bf)], axis=1)
    p1, gate = _proj1(x2, w1, W_igate)

    scale = 1.0 / math.sqrt(DK + DR)
    wq = jnp.concatenate([(W_qc * scale).astype(bf),
                          (W_qr * scale).astype(bf)], axis=1)
    q = _mmb(p1, DCP, 0, wq, bn=768)
    wkv = jnp.concatenate([W_kc.astype(bf), W_v.astype(bf)], axis=1)
    kv = _mmb(p1, DC, 3, wkv, bn=1024)

    cos, sin = _rope_tables()
    qr = _rope(q[:, NH * DK:].reshape(SEQ, NH, DR), cos, sin)
    qr = qr.transpose(1, 0, 2).astype(bf)
    kr = _rope(p1[:, DCP + DC + INH * IHD:DCP + DC + INH * IHD + DR]
               .reshape(SEQ, 1, DR), cos, sin).reshape(SEQ, DR).astype(bf)

    attn = _flash(q, qr, kv, kr, bq=512, bk=512)

    meanh = _indexer(gate.T, p1, W_sq, W_sk, W_sv, W_iout[:, :DV])
    bias = jnp.repeat(meanh, NH // INH, axis=0).reshape(1, NH * DV)

    out = _outproj(attn, bias, W_o)
    return out.reshape(B, SEQ, HID)


# speedup vs baseline: 1.3064x; 1.0233x over previous
import functools
import math

import jax
import jax.numpy as jnp
import numpy as np
from jax import lax
from jax.experimental import pallas as pl
from jax.experimental.pallas import tpu as pltpu

HID = 2048; NH = 16; DK = 128; DR = 64; DV = 128; DC = 512; DCP = 1536
INH = 8; IHD = 128; ITOPK = 8; MAXS = 2048
SEQ = 2048

_NEG = -1e30

_P1N = 3200


def _proj1_kernel(x_ref, w_ref, wg_ref, p_ref, g_ref):
    xb = x_ref[...].astype(jnp.bfloat16)
    p_ref[...] = jnp.dot(xb, w_ref[...],
                         preferred_element_type=jnp.float32
                         ).astype(jnp.bfloat16)

    @pl.when(pl.program_id(0) == 0)
    def _():
        g_ref[...] = jnp.dot(x_ref[...], wg_ref[...],
                             preferred_element_type=jnp.float32)


def _proj1(x2, w1, wg, bn=640):
    return pl.pallas_call(
        _proj1_kernel,
        grid=(_P1N // bn,),
        in_specs=[
            pl.BlockSpec((SEQ, HID), lambda j: (0, 0)),
            pl.BlockSpec((HID, bn), lambda j: (0, j)),
            pl.BlockSpec((HID, INH), lambda j: (0, 0)),
        ],
        out_specs=[
            pl.BlockSpec((SEQ, bn), lambda j: (0, j)),
            pl.BlockSpec((SEQ, INH), lambda j: (0, 0)),
        ],
        out_shape=[
            jax.ShapeDtypeStruct((SEQ, _P1N), jnp.bfloat16),
            jax.ShapeDtypeStruct((SEQ, INH), jnp.float32),
        ],
        compiler_params=pltpu.CompilerParams(
            dimension_semantics=("parallel",)),
    )(x2, w1, wg)


def _mmb_kernel(a_ref, b_ref, o_ref):
    o_ref[...] = jnp.dot(a_ref[...], b_ref[...],
                         preferred_element_type=jnp.float32
                         ).astype(jnp.bfloat16)


def _mmb(a_arr, a_block, a_idx, b, bn):
    K = a_block
    N = b.shape[1]
    return pl.pallas_call(
        _mmb_kernel,
        grid=(N // bn,),
        in_specs=[
            pl.BlockSpec((SEQ, K), lambda j: (0, a_idx)),
            pl.BlockSpec((K, bn), lambda j: (0, j)),
        ],
        out_specs=pl.BlockSpec((SEQ, bn), lambda j: (0, j)),
        out_shape=jax.ShapeDtypeStruct((SEQ, N), jnp.bfloat16),
        compiler_params=pltpu.CompilerParams(
            dimension_semantics=("parallel",)),
    )(a_arr, b)


def _flash_kernel(qc_ref, qr_ref, kc_ref, kr_ref, v_ref, o_ref, *, bq, bk):
    iq = pl.program_id(1)
    qc = qc_ref[...]
    qr = qr_ref[0]

    def blk(j, masked):
        kc = kc_ref[pl.ds(j * bk, bk), :]
        kr = kr_ref[pl.ds(j * bk, bk), :]
        s = lax.dot_general(qc, kc, (((1,), (1,)), ((), ())),
                            preferred_element_type=jnp.float32)
        s += lax.dot_general(qr, kr, (((1,), (1,)), ((), ())),
                             preferred_element_type=jnp.float32)
        if masked:
            row = iq * bq + lax.broadcasted_iota(jnp.int32, (bq, bk), 0)
            col = j * bk + lax.broadcasted_iota(jnp.int32, (bq, bk), 1)
            s = jnp.where(col <= row, s, _NEG)
        return s, v_ref[pl.ds(j * bk, bk), :]

    def update(s, vb, carry):
        m, l, acc = carry
        m_new = jnp.maximum(m, jnp.max(s, axis=1, keepdims=True))
        p = jnp.exp(s - m_new)
        alpha = jnp.exp(m - m_new)
        l_new = l * alpha + jnp.sum(p, axis=1, keepdims=True)
        acc_new = acc * alpha + jnp.dot(p.astype(jnp.bfloat16), vb,
                                        preferred_element_type=jnp.float32)
        return m_new, l_new, acc_new

    def body(j, carry):
        s, vb = blk(j, masked=False)
        return update(s, vb, carry)

    nm = bq // bk
    m0 = jnp.full((bq, 1), _NEG, jnp.float32)
    l0 = jnp.zeros((bq, 1), jnp.float32)
    acc0 = jnp.zeros((bq, DV), jnp.float32)
    carry = lax.fori_loop(0, iq * nm, body, (m0, l0, acc0))
    for t in range(nm):
        s, vb = blk(iq * nm + t, masked=True)
        carry = update(s, vb, carry)
    m, l, acc = carry
    o_ref[...] = (acc / l).astype(jnp.bfloat16)


def _flash(q_arr, qr_rope, kv_arr, kr_rope, bq, bk):
    kern = functools.partial(_flash_kernel, bq=bq, bk=bk)
    return pl.pallas_call(
        kern,
        grid=(NH, SEQ // bq),
        in_specs=[
            pl.BlockSpec((bq, DK), lambda h, i: (i, h)),
            pl.BlockSpec((1, bq, DR), lambda h, i: (h, i, 0)),
            pl.BlockSpec((SEQ, DK), lambda h, i: (0, h)),
            pl.BlockSpec((SEQ, DR), lambda h, i: (0, 0)),
            pl.BlockSpec((SEQ, DV), lambda h, i: (0, NH + h)),
        ],
        out_specs=pl.BlockSpec((bq, DV), lambda h, i: (i, h)),
        out_shape=jax.ShapeDtypeStruct((SEQ, NH * DV), jnp.bfloat16),
        compiler_params=pltpu.CompilerParams(
            dimension_semantics=("parallel", "parallel")),
    )(q_arr, qr_rope, kv_arr, kr_rope, kv_arr)


def _indexer_kernel(g_ref, ip_ref, wq_ref, wk_ref, wv_ref, wo_ref, o_ref):
    g = g_ref[...]
    iota = lax.broadcasted_iota(jnp.int32, (INH, SEQ), 1)
    ip = ip_ref[...]
    dm = (lax.broadcasted_iota(jnp.int32, (INH, INH, 1), 0)
          == lax.broadcasted_iota(jnp.int32, (INH, INH, 1), 1))
    sel_rows = []
    for _ in range(ITOPK):
        m = jnp.max(g, axis=1, keepdims=True)
        cand = jnp.where(g >= m, iota, SEQ)
        idx = jnp.min(cand, axis=1, keepdims=True)
        hot = iota == idx
        onehot = hot.astype(jnp.bfloat16)
        g = jnp.where(hot, _NEG, g)
        full = jnp.dot(onehot, ip, preferred_element_type=jnp.float32)
        f3 = full.reshape(INH, INH, IHD)
        sel_rows.append(jnp.sum(jnp.where(dm, f3, 0.0), axis=1))
    st = jnp.concatenate(sel_rows, axis=0)
    sq = jnp.dot(st, wq_ref[...], preferred_element_type=jnp.float32)
    sk = jnp.dot(st, wk_ref[...], preferred_element_type=jnp.float32)
    sv = jnp.dot(st, wv_ref[...], preferred_element_type=jnp.float32)
    sc = lax.dot_general(sq, sk, (((1,), (1,)), ((), ())),
                         preferred_element_type=jnp.float32)
    sc *= 1.0 / math.sqrt(IHD)
    mx = jnp.max(sc, axis=1, keepdims=True)
    e = jnp.exp(sc - mx)
    p = e / jnp.sum(e, axis=1, keepdims=True)
    so = jnp.dot(p, sv, preferred_element_type=jnp.float32)
    spo = jnp.dot(so, wo_ref[...], preferred_element_type=jnp.float32)
    o_ref[...] = jnp.mean(spo.reshape(ITOPK, INH, DV), axis=0)


def _indexer(gate_t, p1, w_sq, w_sk, w_sv, w_iout_dv):
    return pl.pallas_call(
        _indexer_kernel,
        grid=(1,),
        in_specs=[
            pl.BlockSpec((INH, SEQ), lambda j: (0, 0)),
            pl.BlockSpec((SEQ, INH * IHD), lambda j: (0, 2)),
            pl.BlockSpec((IHD, IHD), lambda j: (0, 0)),
            pl.BlockSpec((IHD, IHD), lambda j: (0, 0)),
            pl.BlockSpec((IHD, IHD), lambda j: (0, 0)),
            pl.BlockSpec((IHD, DV), lambda j: (0, 0)),
        ],
        out_specs=pl.BlockSpec((INH, DV), lambda j: (0, 0)),
        out_shape=jax.ShapeDtypeStruct((INH, DV), jnp.float32),
    )(gate_t, p1, w_sq, w_sk, w_sv, w_iout_dv)


def _out_kernel(a_ref, bias_ref, b_ref, o_ref):
    s = (a_ref[...].astype(jnp.float32) + bias_ref[...]).astype(jnp.bfloat16)
    o_ref[...] = jnp.dot(s, b_ref[...].astype(jnp.bfloat16),
                         preferred_element_type=jnp.float32)


def _outproj(attn, bias, W_o, bn=1024):
    return pl.pallas_call(
        _out_kernel,
        grid=(HID // bn,),
        in_specs=[
            pl.BlockSpec((SEQ, NH * DV), lambda j: (0, 0)),
            pl.BlockSpec((1, NH * DV), lambda j: (0, 0)),
            pl.BlockSpec((NH * DV, bn), lambda j: (0, j)),
        ],
        out_specs=pl.BlockSpec((SEQ, bn), lambda j: (0, j)),
        out_shape=jax.ShapeDtypeStruct((SEQ, HID), jnp.float32),
        compiler_params=pltpu.CompilerParams(
            dimension_semantics=("parallel",)),
    )(attn, bias, W_o)


def _rope_tables():
    inv = 1.0 / (10000.0 ** (np.arange(0, DR, 2)[: DR // 2]
                             .astype(np.float32) / DR))
    t = np.arange(SEQ, dtype=np.float32)
    f = np.outer(t, inv)
    return jnp.asarray(np.cos(f)), jnp.asarray(np.sin(f))


def _rope(x, cos, sin):
    x = x.astype(jnp.float32)
    x1 = x[..., 0::2]
    x2 = x[..., 1::2]
    c = cos[:, None, :]
    s = sin[:, None, :]
    o1 = x1 * c - x2 * s
    o2 = x1 * s + x2 * c
    return jnp.stack([o1, o2], axis=-1).reshape(x.shape)


def kernel(x, W_c, W_cp, W_qc, W_qr, W_kc, W_kr, W_v, W_o,
           W_iproj, W_igate, W_sq, W_sk, W_sv, W_iout):
    B = x.shape[0]
    x2 = x.reshape(B * SEQ, HID)
    bf = jnp.bfloat16

    w1 = jnp.concatenate([
        W_cp.astype(bf), W_c.astype(bf), W_iproj.astype(bf),
        W_kr.astype(bf), jnp.zeros((HID, _P1N - DCP - DC - INH * IHD - DR),
                                   dtype=bf)], axis=1)
    p1, gate = _proj1(x2, w1, W_igate)

    scale = 1.0 / math.sqrt(DK + DR)
    wq = jnp.concatenate([(W_qc * scale).astype(bf),
                          (W_qr * scale).astype(bf)], axis=1)
    q = _mmb(p1, DCP, 0, wq, bn=768)
    wkv = jnp.concatenate([W_kc.astype(bf), W_v.astype(bf)], axis=1)
    kv = _mmb(p1, DC, 3, wkv, bn=1024)

    cos, sin = _rope_tables()
    qr = _rope(q[:, NH * DK:].reshape(SEQ, NH, DR), cos, sin)
    qr = qr.transpose(1, 0, 2).astype(bf)
    kr = _rope(p1[:, DCP + DC + INH * IHD:DCP + DC + INH * IHD + DR]
               .reshape(SEQ, 1, DR), cos, sin).reshape(SEQ, DR).astype(bf)

    attn = _flash(q, qr, kv, kr, bq=1024, bk=512)

    meanh = _indexer(gate.T, p1, W_sq, W_sk, W_sv, W_iout[:, :DV])
    bias = jnp.repeat(meanh, NH // INH, axis=0).reshape(1, NH * DV)

    out = _outproj(attn, bias, W_o)
    return out.reshape(B, SEQ, HID)


# speedup vs baseline: 1.4490x; 1.1091x over previous
import functools
import math

import jax
import jax.numpy as jnp
import numpy as np
from jax import lax
from jax.experimental import pallas as pl
from jax.experimental.pallas import tpu as pltpu

HID = 2048; NH = 16; DK = 128; DR = 64; DV = 128; DC = 512; DCP = 1536
INH = 8; IHD = 128; ITOPK = 8; MAXS = 2048
SEQ = 2048

_NEG = -1e30

_P1N = 3200


def _proj1_kernel(x_ref, w_ref, wg_ref, p_ref, g_ref):
    xb = x_ref[...].astype(jnp.bfloat16)
    p_ref[...] = jnp.dot(xb, w_ref[...],
                         preferred_element_type=jnp.float32
                         ).astype(jnp.bfloat16)

    @pl.when(pl.program_id(0) == 0)
    def _():
        g_ref[...] = jnp.dot(x_ref[...], wg_ref[...],
                             preferred_element_type=jnp.float32)


def _proj1(x2, w1, wg, bn=640):
    return pl.pallas_call(
        _proj1_kernel,
        grid=(_P1N // bn,),
        in_specs=[
            pl.BlockSpec((SEQ, HID), lambda j: (0, 0)),
            pl.BlockSpec((HID, bn), lambda j: (0, j)),
            pl.BlockSpec((HID, INH), lambda j: (0, 0)),
        ],
        out_specs=[
            pl.BlockSpec((SEQ, bn), lambda j: (0, j)),
            pl.BlockSpec((SEQ, INH), lambda j: (0, 0)),
        ],
        out_shape=[
            jax.ShapeDtypeStruct((SEQ, _P1N), jnp.bfloat16),
            jax.ShapeDtypeStruct((SEQ, INH), jnp.float32),
        ],
        compiler_params=pltpu.CompilerParams(
            dimension_semantics=("parallel",)),
    )(x2, w1, wg)


def _mmb_kernel(a_ref, b_ref, o_ref):
    o_ref[...] = jnp.dot(a_ref[...], b_ref[...],
                         preferred_element_type=jnp.float32
                         ).astype(jnp.bfloat16)


def _mmb(a_arr, a_block, a_idx, b, bn):
    K = a_block
    N = b.shape[1]
    return pl.pallas_call(
        _mmb_kernel,
        grid=(N // bn,),
        in_specs=[
            pl.BlockSpec((SEQ, K), lambda j: (0, a_idx)),
            pl.BlockSpec((K, bn), lambda j: (0, j)),
        ],
        out_specs=pl.BlockSpec((SEQ, bn), lambda j: (0, j)),
        out_shape=jax.ShapeDtypeStruct((SEQ, N), jnp.bfloat16),
        compiler_params=pltpu.CompilerParams(
            dimension_semantics=("parallel",)),
    )(a_arr, b)


def _flash_kernel(qc_ref, qr_ref, kc_ref, kr_ref, v_ref, o_ref, *, bq, bk):
    iq = pl.program_id(1)
    qc = qc_ref[...]
    qr = qr_ref[0]

    def blk(j, masked):
        kc = kc_ref[pl.ds(j * bk, bk), :]
        kr = kr_ref[pl.ds(j * bk, bk), :]
        s = lax.dot_general(qc, kc, (((1,), (1,)), ((), ())),
                            preferred_element_type=jnp.float32)
        s += lax.dot_general(qr, kr, (((1,), (1,)), ((), ())),
                             preferred_element_type=jnp.float32)
        if masked:
            row = iq * bq + lax.broadcasted_iota(jnp.int32, (bq, bk), 0)
            col = j * bk + lax.broadcasted_iota(jnp.int32, (bq, bk), 1)
            s = jnp.where(col <= row, s, _NEG)
        return s, v_ref[pl.ds(j * bk, bk), :]

    def update(s, vb, carry):
        m, l, acc = carry
        m_new = jnp.maximum(m, jnp.max(s, axis=1, keepdims=True))
        p = jnp.exp(s - m_new)
        alpha = jnp.exp(m - m_new)
        l_new = l * alpha + jnp.sum(p, axis=1, keepdims=True)
        acc_new = acc * alpha + jnp.dot(p.astype(jnp.bfloat16), vb,
                                        preferred_element_type=jnp.float32)
        return m_new, l_new, acc_new

    def body(j, carry):
        s, vb = blk(j, masked=False)
        return update(s, vb, carry)

    nm = bq // bk
    m0 = jnp.full((bq, 1), _NEG, jnp.float32)
    l0 = jnp.zeros((bq, 1), jnp.float32)
    acc0 = jnp.zeros((bq, DV), jnp.float32)
    carry = lax.fori_loop(0, iq * nm, body, (m0, l0, acc0))
    for t in range(nm):
        s, vb = blk(iq * nm + t, masked=True)
        carry = update(s, vb, carry)
    m, l, acc = carry
    o_ref[...] = (acc / l).astype(jnp.bfloat16)


def _flash(q_arr, qr_rope, kv_arr, kr_rope, bq, bk):
    kern = functools.partial(_flash_kernel, bq=bq, bk=bk)
    return pl.pallas_call(
        kern,
        grid=(NH, SEQ // bq),
        in_specs=[
            pl.BlockSpec((bq, DK), lambda h, i: (i, h)),
            pl.BlockSpec((1, bq, DR), lambda h, i: (h, i, 0)),
            pl.BlockSpec((SEQ, DK), lambda h, i: (0, h)),
            pl.BlockSpec((SEQ, DR), lambda h, i: (0, 0)),
            pl.BlockSpec((SEQ, DV), lambda h, i: (0, NH + h)),
        ],
        out_specs=pl.BlockSpec((bq, DV), lambda h, i: (i, h)),
        out_shape=jax.ShapeDtypeStruct((SEQ, NH * DV), jnp.bfloat16),
        compiler_params=pltpu.CompilerParams(
            dimension_semantics=("parallel", "parallel")),
    )(q_arr, qr_rope, kv_arr, kr_rope, kv_arr)


def _indexer_kernel(g_ref, ip_ref, wq_ref, wk_ref, wv_ref, wo_ref, o_ref):
    g = g_ref[...]
    iota = lax.broadcasted_iota(jnp.int32, (INH, SEQ), 1)
    ip = ip_ref[...]
    dm = (lax.broadcasted_iota(jnp.int32, (INH, INH, 1), 0)
          == lax.broadcasted_iota(jnp.int32, (INH, INH, 1), 1))
    sel_rows = []
    for _ in range(ITOPK):
        m = jnp.max(g, axis=1, keepdims=True)
        cand = jnp.where(g >= m, iota, SEQ)
        idx = jnp.min(cand, axis=1, keepdims=True)
        hot = iota == idx
        onehot = hot.astype(jnp.bfloat16)
        g = jnp.where(hot, _NEG, g)
        full = jnp.dot(onehot, ip, preferred_element_type=jnp.float32)
        f3 = full.reshape(INH, INH, IHD)
        sel_rows.append(jnp.sum(jnp.where(dm, f3, 0.0), axis=1))
    st = jnp.concatenate(sel_rows, axis=0)
    sq = jnp.dot(st, wq_ref[...], preferred_element_type=jnp.float32)
    sk = jnp.dot(st, wk_ref[...], preferred_element_type=jnp.float32)
    sv = jnp.dot(st, wv_ref[...], preferred_element_type=jnp.float32)
    sc = lax.dot_general(sq, sk, (((1,), (1,)), ((), ())),
                         preferred_element_type=jnp.float32)
    sc *= 1.0 / math.sqrt(IHD)
    mx = jnp.max(sc, axis=1, keepdims=True)
    e = jnp.exp(sc - mx)
    p = e / jnp.sum(e, axis=1, keepdims=True)
    so = jnp.dot(p, sv, preferred_element_type=jnp.float32)
    spo = jnp.dot(so, wo_ref[...], preferred_element_type=jnp.float32)
    o_ref[...] = jnp.mean(spo.reshape(ITOPK, INH, DV), axis=0)


def _indexer(gate_t, p1, w_sq, w_sk, w_sv, w_iout_dv):
    return pl.pallas_call(
        _indexer_kernel,
        grid=(1,),
        in_specs=[
            pl.BlockSpec((INH, SEQ), lambda j: (0, 0)),
            pl.BlockSpec((SEQ, INH * IHD), lambda j: (0, 2)),
            pl.BlockSpec((IHD, IHD), lambda j: (0, 0)),
            pl.BlockSpec((IHD, IHD), lambda j: (0, 0)),
            pl.BlockSpec((IHD, IHD), lambda j: (0, 0)),
            pl.BlockSpec((IHD, DV), lambda j: (0, 0)),
        ],
        out_specs=pl.BlockSpec((INH, DV), lambda j: (0, 0)),
        out_shape=jax.ShapeDtypeStruct((INH, DV), jnp.float32),
    )(gate_t, p1, w_sq, w_sk, w_sv, w_iout_dv)


def _out_kernel(a_ref, bias_ref, b_ref, o_ref):
    s = (a_ref[...].astype(jnp.float32) + bias_ref[...]).astype(jnp.bfloat16)
    o_ref[...] = jnp.dot(s, b_ref[...].astype(jnp.bfloat16),
                         preferred_element_type=jnp.float32)


def _outproj(attn, bias, W_o, bn=1024):
    return pl.pallas_call(
        _out_kernel,
        grid=(HID // bn,),
        in_specs=[
            pl.BlockSpec((SEQ, NH * DV), lambda j: (0, 0)),
            pl.BlockSpec((1, NH * DV), lambda j: (0, 0)),
            pl.BlockSpec((NH * DV, bn), lambda j: (0, j)),
        ],
        out_specs=pl.BlockSpec((SEQ, bn), lambda j: (0, j)),
        out_shape=jax.ShapeDtypeStruct((SEQ, HID), jnp.float32),
        compiler_params=pltpu.CompilerParams(
            dimension_semantics=("parallel",)),
    )(attn, bias, W_o)


_HD2 = DR // 2


def _rope_table():
    inv = 1.0 / (10000.0 ** (np.arange(0, DR, 2)[:_HD2]
                             .astype(np.float32) / DR))
    t = np.arange(SEQ, dtype=np.float32)
    f = np.outer(t, inv)
    return jnp.asarray(np.concatenate([np.cos(f), np.sin(f)], axis=1))


def _rope_kernel(qr_ref, kr_ref, tab_ref, qro_ref, kro_ref):
    tab = tab_ref[...]
    c = tab[:, :_HD2]
    s = tab[:, _HD2:]

    def rope(r):
        x1 = r[:, :_HD2]
        x2 = r[:, _HD2:]
        return jnp.concatenate([x1 * c - x2 * s, x1 * s + x2 * c],
                               axis=1).astype(jnp.bfloat16)

    for h in range(NH):
        qro_ref[h] = rope(qr_ref[:, h * DR:(h + 1) * DR].astype(jnp.float32))
    kro_ref[...] = rope(kr_ref[:, :DR].astype(jnp.float32))


def _rope_pallas(q_arr, p1, tab):
    return pl.pallas_call(
        _rope_kernel,
        grid=(1,),
        in_specs=[
            pl.BlockSpec((SEQ, NH * DR), lambda j: (0, NH * DK // (NH * DR))),
            pl.BlockSpec((SEQ, 128), lambda j: (0, (DCP + DC + INH * IHD) // 128)),
            pl.BlockSpec((SEQ, DR), lambda j: (0, 0)),
        ],
        out_specs=[
            pl.BlockSpec((NH, SEQ, DR), lambda j: (0, 0, 0)),
            pl.BlockSpec((SEQ, DR), lambda j: (0, 0)),
        ],
        out_shape=[
            jax.ShapeDtypeStruct((NH, SEQ, DR), jnp.bfloat16),
            jax.ShapeDtypeStruct((SEQ, DR), jnp.bfloat16),
        ],
    )(q_arr, p1, tab)


def kernel(x, W_c, W_cp, W_qc, W_qr, W_kc, W_kr, W_v, W_o,
           W_iproj, W_igate, W_sq, W_sk, W_sv, W_iout):
    B = x.shape[0]
    x2 = x.reshape(B * SEQ, HID)
    bf = jnp.bfloat16

    hperm = np.concatenate([np.arange(0, DR, 2), np.arange(1, DR, 2)])
    qperm = (np.arange(NH)[:, None] * DR + hperm[None, :]).reshape(-1)

    w1 = jnp.concatenate([
        W_cp.astype(bf), W_c.astype(bf), W_iproj.astype(bf),
        W_kr[:, hperm].astype(bf),
        jnp.zeros((HID, _P1N - DCP - DC - INH * IHD - DR),
                  dtype=bf)], axis=1)
    p1, gate = _proj1(x2, w1, W_igate)

    scale = 1.0 / math.sqrt(DK + DR)
    wq = jnp.concatenate([(W_qc * scale).astype(bf),
                          (W_qr[:, qperm] * scale).astype(bf)], axis=1)
    q = _mmb(p1, DCP, 0, wq, bn=768)
    wkv = jnp.concatenate([W_kc.astype(bf), W_v.astype(bf)], axis=1)
    kv = _mmb(p1, DC, 3, wkv, bn=1024)

    qr, kr = _rope_pallas(q, p1, _rope_table())

    attn = _flash(q, qr, kv, kr, bq=1024, bk=512)

    meanh = _indexer(gate.T, p1, W_sq, W_sk, W_sv, W_iout[:, :DV])
    bias = jnp.repeat(meanh, NH // INH, axis=0).reshape(1, NH * DV)

    out = _outproj(attn, bias, W_o)
    return out.reshape(B, SEQ, HID)


# speedup vs baseline: 1.4732x; 1.0167x over previous
import functools
import math

import jax
import jax.numpy as jnp
import numpy as np
from jax import lax
from jax.experimental import pallas as pl
from jax.experimental.pallas import tpu as pltpu

HID = 2048; NH = 16; DK = 128; DR = 64; DV = 128; DC = 512; DCP = 1536
INH = 8; IHD = 128; ITOPK = 8; MAXS = 2048
SEQ = 2048

_NEG = -1e30

_P1N = 3200


def _proj1_kernel(x_ref, w_ref, wg_ref, p_ref, g_ref):
    xb = x_ref[...].astype(jnp.bfloat16)
    p_ref[...] = jnp.dot(xb, w_ref[...],
                         preferred_element_type=jnp.float32
                         ).astype(jnp.bfloat16)

    @pl.when(pl.program_id(0) == 0)
    def _():
        g_ref[...] = jnp.dot(x_ref[...], wg_ref[...],
                             preferred_element_type=jnp.float32)


def _proj1(x2, w1, wg, bn=640):
    return pl.pallas_call(
        _proj1_kernel,
        grid=(_P1N // bn,),
        in_specs=[
            pl.BlockSpec((SEQ, HID), lambda j: (0, 0)),
            pl.BlockSpec((HID, bn), lambda j: (0, j)),
            pl.BlockSpec((HID, INH), lambda j: (0, 0)),
        ],
        out_specs=[
            pl.BlockSpec((SEQ, bn), lambda j: (0, j)),
            pl.BlockSpec((SEQ, INH), lambda j: (0, 0)),
        ],
        out_shape=[
            jax.ShapeDtypeStruct((SEQ, _P1N), jnp.bfloat16),
            jax.ShapeDtypeStruct((SEQ, INH), jnp.float32),
        ],
        compiler_params=pltpu.CompilerParams(
            dimension_semantics=("parallel",)),
    )(x2, w1, wg)


def _mmb_kernel(a_ref, b_ref, o_ref):
    o_ref[...] = jnp.dot(a_ref[...], b_ref[...],
                         preferred_element_type=jnp.float32
                         ).astype(jnp.bfloat16)


def _mmb(a_arr, a_block, a_idx, b, bn):
    K = a_block
    N = b.shape[1]
    return pl.pallas_call(
        _mmb_kernel,
        grid=(N // bn,),
        in_specs=[
            pl.BlockSpec((SEQ, K), lambda j: (0, a_idx)),
            pl.BlockSpec((K, bn), lambda j: (0, j)),
        ],
        out_specs=pl.BlockSpec((SEQ, bn), lambda j: (0, j)),
        out_shape=jax.ShapeDtypeStruct((SEQ, N), jnp.bfloat16),
        compiler_params=pltpu.CompilerParams(
            dimension_semantics=("parallel",)),
    )(a_arr, b)


def _flash_kernel(qc_ref, qr_ref, kc_ref, kr_ref, v_ref, o_ref, *, bq, bk):
    iq = pl.program_id(1)
    qc = qc_ref[...]
    qr = qr_ref[0]

    def blk(j, masked):
        kc = kc_ref[pl.ds(j * bk, bk), :]
        kr = kr_ref[pl.ds(j * bk, bk), :]
        s = lax.dot_general(qc, kc, (((1,), (1,)), ((), ())),
                            preferred_element_type=jnp.float32)
        s += lax.dot_general(qr, kr, (((1,), (1,)), ((), ())),
                             preferred_element_type=jnp.float32)
        if masked:
            row = iq * bq + lax.broadcasted_iota(jnp.int32, (bq, bk), 0)
            col = j * bk + lax.broadcasted_iota(jnp.int32, (bq, bk), 1)
            s = jnp.where(col <= row, s, _NEG)
        return s, v_ref[pl.ds(j * bk, bk), :]

    def update(s, vb, carry):
        m, l, acc = carry
        m_new = jnp.maximum(m, jnp.max(s, axis=1, keepdims=True))
        p = jnp.exp2(s - m_new)
        alpha = jnp.exp2(m - m_new)
        l_new = l * alpha + jnp.sum(p, axis=1, keepdims=True)
        acc_new = acc * alpha + jnp.dot(p.astype(jnp.bfloat16), vb,
                                        preferred_element_type=jnp.float32)
        return m_new, l_new, acc_new

    def body(j, carry):
        s, vb = blk(j, masked=False)
        return update(s, vb, carry)

    nm = bq // bk
    m0 = jnp.full((bq, 1), _NEG, jnp.float32)
    l0 = jnp.zeros((bq, 1), jnp.float32)
    acc0 = jnp.zeros((bq, DV), jnp.float32)
    carry = lax.fori_loop(0, iq * nm, body, (m0, l0, acc0))
    for t in range(nm):
        s, vb = blk(iq * nm + t, masked=True)
        carry = update(s, vb, carry)
    m, l, acc = carry
    o_ref[...] = (acc / l).astype(jnp.bfloat16)


def _flash(q_arr, qr_rope, kv_arr, kr_rope, bq, bk):
    kern = functools.partial(_flash_kernel, bq=bq, bk=bk)
    return pl.pallas_call(
        kern,
        grid=(NH, SEQ // bq),
        in_specs=[
            pl.BlockSpec((bq, DK), lambda h, i: (i, h)),
            pl.BlockSpec((1, bq, DR), lambda h, i: (h, i, 0)),
            pl.BlockSpec((SEQ, DK), lambda h, i: (0, h)),
            pl.BlockSpec((SEQ, DR), lambda h, i: (0, 0)),
            pl.BlockSpec((SEQ, DV), lambda h, i: (0, NH + h)),
        ],
        out_specs=pl.BlockSpec((bq, DV), lambda h, i: (i, h)),
        out_shape=jax.ShapeDtypeStruct((SEQ, NH * DV), jnp.bfloat16),
        compiler_params=pltpu.CompilerParams(
            dimension_semantics=("parallel", "parallel")),
    )(q_arr, qr_rope, kv_arr, kr_rope, kv_arr)


def _indexer_kernel(g_ref, ip_ref, wq_ref, wk_ref, wv_ref, wo_ref, o_ref):
    g = g_ref[...]
    iota = lax.broadcasted_iota(jnp.int32, (INH, SEQ), 1)
    ip = ip_ref[...]
    dm = (lax.broadcasted_iota(jnp.int32, (INH, INH, 1), 0)
          == lax.broadcasted_iota(jnp.int32, (INH, INH, 1), 1))
    sel_rows = []
    for _ in range(ITOPK):
        m = jnp.max(g, axis=1, keepdims=True)
        cand = jnp.where(g >= m, iota, SEQ)
        idx = jnp.min(cand, axis=1, keepdims=True)
        hot = iota == idx
        onehot = hot.astype(jnp.bfloat16)
        g = jnp.where(hot, _NEG, g)
        full = jnp.dot(onehot, ip, preferred_element_type=jnp.float32)
        f3 = full.reshape(INH, INH, IHD)
        sel_rows.append(jnp.sum(jnp.where(dm, f3, 0.0), axis=1))
    st = jnp.concatenate(sel_rows, axis=0)
    sq = jnp.dot(st, wq_ref[...], preferred_element_type=jnp.float32)
    sk = jnp.dot(st, wk_ref[...], preferred_element_type=jnp.float32)
    sv = jnp.dot(st, wv_ref[...], preferred_element_type=jnp.float32)
    sc = lax.dot_general(sq, sk, (((1,), (1,)), ((), ())),
                         preferred_element_type=jnp.float32)
    sc *= 1.0 / math.sqrt(IHD)
    mx = jnp.max(sc, axis=1, keepdims=True)
    e = jnp.exp(sc - mx)
    p = e / jnp.sum(e, axis=1, keepdims=True)
    so = jnp.dot(p, sv, preferred_element_type=jnp.float32)
    spo = jnp.dot(so, wo_ref[...], preferred_element_type=jnp.float32)
    o_ref[...] = jnp.mean(spo.reshape(ITOPK, INH, DV), axis=0)


def _indexer(gate_t, p1, w_sq, w_sk, w_sv, w_iout_dv):
    return pl.pallas_call(
        _indexer_kernel,
        grid=(1,),
        in_specs=[
            pl.BlockSpec((INH, SEQ), lambda j: (0, 0)),
            pl.BlockSpec((SEQ, INH * IHD), lambda j: (0, 2)),
            pl.BlockSpec((IHD, IHD), lambda j: (0, 0)),
            pl.BlockSpec((IHD, IHD), lambda j: (0, 0)),
            pl.BlockSpec((IHD, IHD), lambda j: (0, 0)),
            pl.BlockSpec((IHD, DV), lambda j: (0, 0)),
        ],
        out_specs=pl.BlockSpec((INH, DV), lambda j: (0, 0)),
        out_shape=jax.ShapeDtypeStruct((INH, DV), jnp.float32),
    )(gate_t, p1, w_sq, w_sk, w_sv, w_iout_dv)


def _out_kernel(a_ref, bias_ref, b_ref, o_ref):
    s = (a_ref[...].astype(jnp.float32) + bias_ref[...]).astype(jnp.bfloat16)
    o_ref[...] = jnp.dot(s, b_ref[...].astype(jnp.bfloat16),
                         preferred_element_type=jnp.float32)


def _outproj(attn, bias, W_o, bn=1024):
    return pl.pallas_call(
        _out_kernel,
        grid=(HID // bn,),
        in_specs=[
            pl.BlockSpec((SEQ, NH * DV), lambda j: (0, 0)),
            pl.BlockSpec((1, NH * DV), lambda j: (0, 0)),
            pl.BlockSpec((NH * DV, bn), lambda j: (0, j)),
        ],
        out_specs=pl.BlockSpec((SEQ, bn), lambda j: (0, j)),
        out_shape=jax.ShapeDtypeStruct((SEQ, HID), jnp.float32),
        compiler_params=pltpu.CompilerParams(
            dimension_semantics=("parallel",)),
    )(attn, bias, W_o)


_HD2 = DR // 2


def _rope_table():
    inv = 1.0 / (10000.0 ** (np.arange(0, DR, 2)[:_HD2]
                             .astype(np.float32) / DR))
    t = np.arange(SEQ, dtype=np.float32)
    f = np.outer(t, inv)
    return jnp.asarray(np.concatenate([np.cos(f), np.sin(f)], axis=1))


def _rope_kernel(qr_ref, kr_ref, tab_ref, qro_ref, kro_ref):
    tab = tab_ref[...]
    c = tab[:, :_HD2]
    s = tab[:, _HD2:]

    def rope(r):
        x1 = r[:, :_HD2]
        x2 = r[:, _HD2:]
        return jnp.concatenate([x1 * c - x2 * s, x1 * s + x2 * c],
                               axis=1).astype(jnp.bfloat16)

    for h in range(NH):
        qro_ref[h] = rope(qr_ref[:, h * DR:(h + 1) * DR].astype(jnp.float32))
    kro_ref[...] = rope(kr_ref[:, :DR].astype(jnp.float32))


def _rope_pallas(q_arr, p1, tab):
    return pl.pallas_call(
        _rope_kernel,
        grid=(1,),
        in_specs=[
            pl.BlockSpec((SEQ, NH * DR), lambda j: (0, NH * DK // (NH * DR))),
            pl.BlockSpec((SEQ, 128), lambda j: (0, (DCP + DC + INH * IHD) // 128)),
            pl.BlockSpec((SEQ, DR), lambda j: (0, 0)),
        ],
        out_specs=[
            pl.BlockSpec((NH, SEQ, DR), lambda j: (0, 0, 0)),
            pl.BlockSpec((SEQ, DR), lambda j: (0, 0)),
        ],
        out_shape=[
            jax.ShapeDtypeStruct((NH, SEQ, DR), jnp.bfloat16),
            jax.ShapeDtypeStruct((SEQ, DR), jnp.bfloat16),
        ],
    )(q_arr, p1, tab)


def kernel(x, W_c, W_cp, W_qc, W_qr, W_kc, W_kr, W_v, W_o,
           W_iproj, W_igate, W_sq, W_sk, W_sv, W_iout):
    B = x.shape[0]
    x2 = x.reshape(B * SEQ, HID)
    bf = jnp.bfloat16

    hperm = np.concatenate([np.arange(0, DR, 2), np.arange(1, DR, 2)])
    qperm = (np.arange(NH)[:, None] * DR + hperm[None, :]).reshape(-1)

    w1 = jnp.concatenate([
        W_cp.astype(bf), W_c.astype(bf), W_iproj.astype(bf),
        W_kr[:, hperm].astype(bf),
        jnp.zeros((HID, _P1N - DCP - DC - INH * IHD - DR),
                  dtype=bf)], axis=1)
    p1, gate = _proj1(x2, w1, W_igate)

    scale = math.log2(math.e) / math.sqrt(DK + DR)
    wq = jnp.concatenate([(W_qc * scale).astype(bf),
                          (W_qr[:, qperm] * scale).astype(bf)], axis=1)
    q = _mmb(p1, DCP, 0, wq, bn=768)
    wkv = jnp.concatenate([W_kc.astype(bf), W_v.astype(bf)], axis=1)
    kv = _mmb(p1, DC, 3, wkv, bn=1024)

    qr, kr = _rope_pallas(q, p1, _rope_table())

    attn = _flash(q, qr, kv, kr, bq=1024, bk=512)

    meanh = _indexer(gate.T, p1, W_sq, W_sk, W_sv, W_iout[:, :DV])
    bias = jnp.repeat(meanh, NH // INH, axis=0).reshape(1, NH * DV)

    out = _outproj(attn, bias, W_o)
    return out.reshape(B, SEQ, HID)
